# Initial kernel scaffold; baseline (speedup 1.0000x reference)
import jax
import jax.numpy as jnp
from jax.experimental import pallas as pl


def kernel(anchor, gt):
    raise NotImplementedError("write your pallas kernel here")


# speedup vs baseline: 72.4302x; 72.4302x over previous
import jax
import jax.numpy as jnp
from jax.experimental import pallas as pl
from jax.experimental.pallas import tpu as pltpu

_POS = 0.5
_NEG = 0.3
_LOW = 0.1
_N_PAD = 20480
_BIG = 2**30


def _body(a_ref, g_ref, out_ref):
    a = a_ref[...]
    g = g_ref[...]

    ax0 = a[0:1, :]
    ay0 = a[1:2, :]
    ax1 = a[2:3, :]
    ay1 = a[3:4, :]
    gx0 = g[:, 0:1]
    gy0 = g[:, 1:2]
    gx1 = g[:, 2:3]
    gy1 = g[:, 3:4]

    area_a = (ax1 - ax0) * (ay1 - ay0)
    area_b = (gx1 - gx0) * (gy1 - gy0)

    ltx = jnp.maximum(ax0, gx0)
    lty = jnp.maximum(ay0, gy0)
    rbx = jnp.minimum(ax1, gx1)
    rby = jnp.minimum(ay1, gy1)
    w = jnp.maximum(rbx - ltx, 0.0)
    h = jnp.maximum(rby - lty, 0.0)
    inter = w * h
    union = area_a + area_b - inter
    iou = inter / jnp.maximum(union, 1e-9)

    row = jax.lax.broadcasted_iota(jnp.int32, iou.shape, 0)
    col = jax.lax.broadcasted_iota(jnp.int32, iou.shape, 1)

    maxi = jnp.max(iou, axis=0, keepdims=True)
    amax = jnp.min(jnp.where(iou == maxi, row, _BIG), axis=0,
                   keepdims=True)
    base = jnp.where(maxi > _POS, amax, jnp.int32(-2))
    base = jnp.where(maxi < _NEG, jnp.int32(-1), base)

    m1 = jnp.max(iou, axis=1, keepdims=True)
    i1 = jnp.min(jnp.where(iou == m1, col, _BIG), axis=1, keepdims=True)
    iou2 = jnp.where(col == i1, -1.0, iou)
    m2 = jnp.max(iou2, axis=1, keepdims=True)
    i2 = jnp.min(jnp.where(iou2 == m2, col, _BIG), axis=1, keepdims=True)
    iou3 = jnp.where(col == i2, -1.0, iou2)
    m3 = jnp.max(iou3, axis=1, keepdims=True)
    i3 = jnp.min(jnp.where(iou3 == m3, col, _BIG), axis=1, keepdims=True)

    npos = ((m1 > _POS).astype(jnp.int32) + (m2 > _POS).astype(jnp.int32)
            + (m3 > _POS).astype(jnp.int32))
    cond = npos < 3

    w2 = (m2 > _LOW) & cond
    w3 = (m3 > _LOW) & cond
    hit = ((col == i1)
           | ((col == i2) & w2)
           | ((col == i3) & w3))

    gwin = jnp.max(jnp.where(hit, row, jnp.int32(-1)), axis=0,
                   keepdims=True)
    out_ref[...] = jnp.where(gwin >= 0, gwin, base)


def kernel(anchor, gt):
    n = anchor.shape[0]
    pad = _N_PAD - n
    a_t = jnp.pad(anchor, ((0, pad), (0, 0))).T
    out = pl.pallas_call(
        _body,
        out_shape=jax.ShapeDtypeStruct((1, _N_PAD), jnp.int32),
    )(a_t, gt)
    return out.reshape(_N_PAD)[:n]
